# Initial kernel scaffold; baseline (speedup 1.0000x reference)
import jax
import jax.numpy as jnp
from jax.experimental import pallas as pl


def kernel(x, edge_index, Ws0, Wn0, b0, Ws1, Wn1, b1, Ws2, Wn2, b2, pw1, pb1, pw2, pb2, pw3, pb3):
    raise NotImplementedError("write your pallas kernel here")


# speedup vs baseline: 3.8597x; 3.8597x over previous
import functools

import jax
import jax.numpy as jnp
from jax import lax
from jax.experimental import pallas as pl
from jax.experimental.pallas import tpu as pltpu
from jax.experimental.pallas import tpu_sc as plsc

N = 10002
E = 320064
D = 128
N_PAD = 10240
NC = 2
NS = 16
NW = NC * NS
CHUNK = 128
C = 79
E_PAD = NW * C * CHUNK
ROWS_PER_TILE = N_PAD // NS

@functools.lru_cache(maxsize=None)
def _make_segsum(with_deg: bool):
    mesh = plsc.VectorSubcoreMesh(
        core_axis_name="c", subcore_axis_name="s",
        num_cores=NC, num_subcores=NS)
    out_type = [jax.ShapeDtypeStruct((NC, N_PAD, D), jnp.float32)]
    scratch = [
        pltpu.VMEM((2, CHUNK), jnp.int32),
        pltpu.VMEM((CHUNK, D), jnp.float32),
        pltpu.VMEM_SHARED((N_PAD, D), jnp.float32),
        pltpu.SemaphoreType.DMA,
    ]
    if with_deg:
        out_type.append(jax.ShapeDtypeStruct((NC, N_PAD, 16), jnp.float32))
        scratch.append(pltpu.VMEM((CHUNK, 16), jnp.float32))
        scratch.append(pltpu.VMEM_SHARED((N_PAD, 16), jnp.float32))

    def body(*refs):
        if with_deg:
            (h_hbm, idx_hbm, zeros_hbm, zeros8_hbm, ones_hbm,
             out_hbm, degout_hbm,
             idx_v, rows_v, accum_sh, sem, ones_v, deg_sh) = refs
        else:
            (h_hbm, idx_hbm, zeros_hbm,
             out_hbm,
             idx_v, rows_v, accum_sh, sem) = refs
        c = lax.axis_index("c")
        s = lax.axis_index("s")
        wid = c * NS + s
        r0 = s * ROWS_PER_TILE

        pltpu.sync_copy(zeros_hbm.at[pl.ds(r0, ROWS_PER_TILE)],
                        accum_sh.at[pl.ds(r0, ROWS_PER_TILE)])
        if with_deg:
            pltpu.sync_copy(zeros8_hbm.at[pl.ds(r0, ROWS_PER_TILE)],
                            deg_sh.at[pl.ds(r0, ROWS_PER_TILE)])
            pltpu.sync_copy(ones_hbm, ones_v)
        plsc.subcore_barrier()

        @pl.loop(0, C)
        def _chunk(j):
            pltpu.sync_copy(idx_hbm.at[wid, j], idx_v)
            pltpu.async_copy(h_hbm.at[idx_v.at[0]], rows_v, sem).wait()
            pltpu.sync_copy(rows_v, accum_sh.at[idx_v.at[1]], add=True)
            if with_deg:
                pltpu.sync_copy(ones_v, deg_sh.at[idx_v.at[1]], add=True)

        plsc.subcore_barrier()
        pltpu.sync_copy(accum_sh.at[pl.ds(r0, ROWS_PER_TILE)],
                        out_hbm.at[c, pl.ds(r0, ROWS_PER_TILE)])
        if with_deg:
            pltpu.sync_copy(deg_sh.at[pl.ds(r0, ROWS_PER_TILE)],
                            degout_hbm.at[c, pl.ds(r0, ROWS_PER_TILE)])

    return pl.kernel(body, out_type=tuple(out_type), mesh=mesh,
                     scratch_types=scratch,
                     compiler_params=pltpu.CompilerParams(
                         use_tc_tiling_on_sc=False))


_ROW_BLK = 512


def _layer_call(h, p0, p1, d0, d1, Ws, Wn, b, relu):
    def body(h_ref, p0_ref, p1_ref, d0_ref, d1_ref, ws_ref, wn_ref, b_ref,
             o_ref):
        deg = d0_ref[:, 0:1] + d1_ref[:, 0:1]
        rdeg = 1.0 / jnp.maximum(deg, 1.0)
        hn = (p0_ref[...] + p1_ref[...]) * rdeg
        acc = (jnp.dot(h_ref[...], ws_ref[...],
                       preferred_element_type=jnp.float32)
               + jnp.dot(hn, wn_ref[...], preferred_element_type=jnp.float32)
               + b_ref[...])
        o_ref[...] = jnp.maximum(acc, 0.0) if relu else acc

    grid = (N_PAD // _ROW_BLK,)
    row = lambda i: (i, 0)
    fixed = lambda i: (0, 0)
    return pl.pallas_call(
        body,
        grid=grid,
        in_specs=[
            pl.BlockSpec((_ROW_BLK, D), row),
            pl.BlockSpec((_ROW_BLK, D), row),
            pl.BlockSpec((_ROW_BLK, D), row),
            pl.BlockSpec((_ROW_BLK, 16), row),
            pl.BlockSpec((_ROW_BLK, 16), row),
            pl.BlockSpec((D, D), fixed),
            pl.BlockSpec((D, D), fixed),
            pl.BlockSpec((1, D), fixed),
        ],
        out_specs=pl.BlockSpec((_ROW_BLK, D), row),
        out_shape=jax.ShapeDtypeStruct((N_PAD, D), jnp.float32),
    )(h, p0, p1, d0, d1, Ws, Wn, b)


_R_PAD = 3336


def _predictor_call(sh, ph, nh, pw1, pb1, pw2, pb2, pw3, pb3):
    def body(s_ref, p_ref, n_ref, w1_ref, b1_ref, w2_ref, b2_ref, w3_ref,
             b3_ref, op_ref, on_ref):
        w1 = w1_ref[...]
        w2 = w2_ref[...]
        w3 = w3_ref[...]
        for z_in, o_ref in ((s_ref[...] * p_ref[...], op_ref),
                            (s_ref[...] * n_ref[...], on_ref)):
            z = jnp.maximum(
                jnp.dot(z_in, w1, preferred_element_type=jnp.float32)
                + b1_ref[...], 0.0)
            z = jnp.maximum(
                jnp.dot(z, w2, preferred_element_type=jnp.float32)
                + b2_ref[...], 0.0)
            o_ref[...] = (jnp.dot(z, w3, preferred_element_type=jnp.float32)
                          + b3_ref[...])

    return pl.pallas_call(
        body,
        out_shape=(jax.ShapeDtypeStruct((_R_PAD, 1), jnp.float32),
                   jax.ShapeDtypeStruct((_R_PAD, 1), jnp.float32)),
    )(sh, ph, nh, pw1, pb1.reshape(1, D), pw2, pb2.reshape(1, D), pw3,
      pb3.reshape(1, 1))


def kernel(x, edge_index, Ws0, Wn0, b0, Ws1, Wn1, b1, Ws2, Wn2, b2,
           pw1, pb1, pw2, pb2, pw3, pb3):
    src = edge_index[0]
    dst = edge_index[1]
    pad = E_PAD - E
    srcp = jnp.concatenate(
        [src, jnp.zeros((pad,), jnp.int32)]).reshape(NW, C, 1, CHUNK)
    dstp = jnp.concatenate(
        [dst, jnp.full((pad,), N_PAD - 1, jnp.int32)]).reshape(NW, C, 1, CHUNK)
    idxp = jnp.concatenate([srcp, dstp], axis=2)

    h = jnp.pad(x, ((0, N_PAD - N), (0, 0)))
    zeros = jnp.zeros((N_PAD, D), jnp.float32)
    zeros8 = jnp.zeros((N_PAD, 16), jnp.float32)
    ones = jnp.ones((CHUNK, 16), jnp.float32)

    p, dp = _make_segsum(True)(h, idxp, zeros, zeros8, ones)
    h = _layer_call(h, p[0], p[1], dp[0], dp[1], Ws0, Wn0,
                    b0.reshape(1, D), True)
    (p,) = _make_segsum(False)(h, idxp, zeros)
    h = _layer_call(h, p[0], p[1], dp[0], dp[1], Ws1, Wn1,
                    b1.reshape(1, D), True)
    (p,) = _make_segsum(False)(h, idxp, zeros)
    h = _layer_call(h, p[0], p[1], dp[0], dp[1], Ws2, Wn2,
                    b2.reshape(1, D), False)

    third = N // 3
    sh = jnp.pad(h[0:third], ((0, _R_PAD - third), (0, 0)))
    ph = jnp.pad(h[third:2 * third], ((0, _R_PAD - third), (0, 0)))
    nh = jnp.pad(h[2 * third:N], ((0, _R_PAD - third), (0, 0)))
    h_pos, h_neg = _predictor_call(sh, ph, nh, pw1, pb1, pw2, pb2, pw3, pb3)
    return (h_pos[:third], h_neg[:third])
